# baseline (speedup 1.0000x reference)
import functools

import jax
import jax.numpy as jnp
from jax import lax
from jax.experimental import pallas as pl
from jax.experimental.pallas import tpu as pltpu
from jax.experimental.pallas import tpu_sc as plsc

N_SHIFT = 128


def kernel(x):
    B, R, T = x.shape
    rows = B * R
    n_workers = 32
    rows_per_w = rows // n_workers
    body = T - N_SHIFT

    mesh = plsc.VectorSubcoreMesh(core_axis_name="c", subcore_axis_name="s")

    @functools.partial(
        pl.kernel,
        mesh=mesh,
        out_type=jax.ShapeDtypeStruct((rows, T), jnp.float32),
    )
    def sc_shift(x_hbm, out_hbm):
        c = lax.axis_index("c")
        s = lax.axis_index("s")
        wid = s * 2 + c
        base = wid * rows_per_w
        pltpu.sync_copy(
            x_hbm.at[pl.ds(base, rows_per_w), pl.ds(0, body)],
            out_hbm.at[pl.ds(base, rows_per_w), pl.ds(N_SHIFT, body)],
        )
        pltpu.sync_copy(
            x_hbm.at[pl.ds(base, rows_per_w), pl.ds(body, N_SHIFT)],
            out_hbm.at[pl.ds(base, rows_per_w), pl.ds(0, N_SHIFT)],
        )

    out = sc_shift(x.reshape(rows, T))
    return out.reshape(B, R, T)


# speedup vs baseline: 49.4638x; 49.4638x over previous
import jax
import jax.numpy as jnp
from jax.experimental import pallas as pl
from jax.experimental.pallas import tpu as pltpu

N_SHIFT = 128
BS = 256


def _body(x_ref, o_ref):
    o_ref[:, N_SHIFT:] = x_ref[:, : o_ref.shape[1] - N_SHIFT]
    o_ref[:, :N_SHIFT] = x_ref[:, o_ref.shape[1] - N_SHIFT :]


def kernel(x):
    B, R, T = x.shape
    rows = B * R
    xf = x.reshape(rows, T)
    out = pl.pallas_call(
        _body,
        grid=(rows // BS,),
        in_specs=[pl.BlockSpec((BS, T), lambda i: (i, 0))],
        out_specs=pl.BlockSpec((BS, T), lambda i: (i, 0)),
        out_shape=jax.ShapeDtypeStruct((rows, T), jnp.float32),
    )(xf)
    return out.reshape(B, R, T)
